# Initial kernel scaffold; baseline (speedup 1.0000x reference)
import jax
import jax.numpy as jnp
from jax.experimental import pallas as pl


def kernel(x, edge_index, W1, al1, ar1, W2, al2, ar2, Wg1, bg1, Wg2, bg2, Wg3, bg3, Wg4, bg4, Wc13, b13, Wc31, b31, Wm1, bm1, Wm2, bm2, Wf1, bf1, Wf2, bf2, Wa1, ba1, Wa2, ba2, ws, bs):
    raise NotImplementedError("write your pallas kernel here")


# speedup vs baseline: 27.0126x; 27.0126x over previous
import functools

import jax
import jax.numpy as jnp
from jax import lax
from jax.experimental import pallas as pl
from jax.experimental.pallas import tpu as pltpu
from jax.experimental.pallas import tpu_sc as plsc

N, E, D = 10000, 320000, 128
H, Dh = 2, 64
NEG = 0.2
K = int(0.5 * 0.9 * N)

NW = 32
CH = 64
CHUNKS = 157
EPW = CH * CHUNKS
EPAD = NW * EPW
NP = N + 8
WT = 10
RPT = N // WT
RB = 40
ZD = 2000

KP = 4608
PPW = KP // NW
PC = 72


def _dense1_body(x_ref, W1_ref,
                 Wg1_ref, bg1_ref, Wg2_ref, bg2_ref, Wg3_ref, bg3_ref,
                 Wg4_ref, bg4_ref, feat_ref, chan_ref):
    x = x_ref[...]
    feat = jnp.dot(x, W1_ref[...], preferred_element_type=jnp.float32)
    feat_ref[...] = feat
    c = jnp.dot(jax.nn.relu(jnp.dot(x, Wg1_ref[...], preferred_element_type=jnp.float32) + bg1_ref[...]),
                Wg2_ref[...], preferred_element_type=jnp.float32) + bg2_ref[...]
    mx = jnp.max(c, axis=1, keepdims=True)
    av = jnp.mean(c, axis=1, keepdims=True)
    comb = jax.nn.sigmoid(mx) * c + jax.nn.sigmoid(av) * c
    chan = jnp.dot(jax.nn.relu(jnp.dot(comb, Wg3_ref[...], preferred_element_type=jnp.float32) + bg3_ref[...]),
                   Wg4_ref[...], preferred_element_type=jnp.float32) + bg4_ref[...]
    chan_ref[...] = chan


def _elr_body(feat_ref, al0_ref, al1_ref, ar0_ref, ar1_ref, elr_ref):
    feat = feat_ref[...]
    f0 = feat[:, :Dh]
    f1 = feat[:, Dh:]
    el0 = jnp.sum(f0 * al0_ref[...], axis=1)
    el1 = jnp.sum(f1 * al1_ref[...], axis=1)
    er0 = jnp.sum(f0 * ar0_ref[...], axis=1)
    er1 = jnp.sum(f1 * ar1_ref[...], axis=1)
    elr_ref[...] = jnp.stack([el0, el1, er0, er1], axis=0)


def _elr(feat, al, ar):
    return pl.pallas_call(
        _elr_body,
        out_shape=jax.ShapeDtypeStruct((4, N), jnp.float32),
    )(feat, al[0, 0][None, :], al[0, 1][None, :], ar[0, 0][None, :], ar[0, 1][None, :])


def _edge_body(feat_hbm, el0_hbm, el1_hbm, er0_hbm, er1_hbm, src_hbm, dst_hbm,
               acc_out, den_out,
               acc_sp, den0_sp, den1_sp,
               srcb, dstb, el0c, el1c, er0c, er1c, rows, ee0b, ee1b, zb, zd):
    c = lax.axis_index("c")
    s = lax.axis_index("s")
    wid = s * 2 + c

    def _zrow(i, _):
        for k in range(8):
            zb[i, pl.ds(k * 16, 16)] = jnp.zeros((16,), jnp.float32)
        return 0
    lax.fori_loop(0, RB, _zrow, 0)

    def _zd16(i, _):
        zd[pl.ds(i * 16, 16)] = jnp.zeros((16,), jnp.float32)
        return 0
    lax.fori_loop(0, ZD // 16, _zd16, 0)

    @pl.when(s < WT)
    def _zero_acc():
        def _zi(i, _):
            pltpu.sync_copy(zb, acc_sp.at[pl.ds(s * RPT + i * RB, RB)])
            return 0
        lax.fori_loop(0, RPT // RB, _zi, 0)
        pltpu.sync_copy(zd.at[pl.ds(0, RPT)], den0_sp.at[pl.ds(s * RPT, RPT)])
        pltpu.sync_copy(zd.at[pl.ds(0, RPT)], den1_sp.at[pl.ds(s * RPT, RPT)])

    plsc.subcore_barrier()

    def _chunk(ch, _):
        off = wid * EPW + ch * CH
        pltpu.sync_copy(src_hbm.at[pl.ds(off, CH)], srcb)
        pltpu.sync_copy(dst_hbm.at[pl.ds(off, CH)], dstb)
        pltpu.sync_copy(el0_hbm.at[srcb], el0c)
        pltpu.sync_copy(el1_hbm.at[srcb], el1c)
        pltpu.sync_copy(er0_hbm.at[dstb], er0c)
        pltpu.sync_copy(er1_hbm.at[dstb], er1c)
        pltpu.sync_copy(feat_hbm.at[srcb], rows)

        def _grp(g, _):
            sl16 = pl.ds(g * 16, 16)
            e0 = el0c[sl16] + er0c[sl16]
            e0 = jnp.where(e0 >= 0, e0, NEG * e0)
            ee0b[sl16] = jnp.exp(e0)
            e1 = el1c[sl16] + er1c[sl16]
            e1 = jnp.where(e1 >= 0, e1, NEG * e1)
            ee1b[sl16] = jnp.exp(e1)
            return 0
        lax.fori_loop(0, CH // 16, _grp, 0)

        def _scale(g, _):
            v0 = ee0b[pl.ds(g * 16, 16)]
            v1 = ee1b[pl.ds(g * 16, 16)]
            for j in range(16):
                s0 = v0[j]
                s1 = v1[j]
                for k in range(4):
                    sl = pl.ds(k * 16, 16)
                    rows[g * 16 + j, sl] = rows[g * 16 + j, sl] * s0
                for k in range(4, 8):
                    sl = pl.ds(k * 16, 16)
                    rows[g * 16 + j, sl] = rows[g * 16 + j, sl] * s1
            return 0
        lax.fori_loop(0, CH // 16, _scale, 0)

        pltpu.sync_copy(rows, acc_sp.at[dstb], add=True)
        pltpu.sync_copy(ee0b, den0_sp.at[dstb], add=True)
        pltpu.sync_copy(ee1b, den1_sp.at[dstb], add=True)
        return 0
    lax.fori_loop(0, CHUNKS, _chunk, 0)

    plsc.subcore_barrier()

    @pl.when(s < WT)
    def _write_acc():
        def _wi(i, _):
            r0 = s * RPT + i * RB
            pltpu.sync_copy(acc_sp.at[pl.ds(r0, RB)], zb)
            pltpu.sync_copy(zb, acc_out.at[c, pl.ds(r0, RB)])
            return 0
        lax.fori_loop(0, RPT // RB, _wi, 0)
        pltpu.sync_copy(den0_sp.at[pl.ds(s * RPT, RPT)], zd.at[pl.ds(0, RPT)])
        pltpu.sync_copy(zd.at[pl.ds(0, RPT)],
                        den_out.at[pl.ds(c * 2 * N + s * RPT, RPT)])
        pltpu.sync_copy(den1_sp.at[pl.ds(s * RPT, RPT)], zd.at[pl.ds(0, RPT)])
        pltpu.sync_copy(zd.at[pl.ds(0, RPT)],
                        den_out.at[pl.ds(c * 2 * N + N + s * RPT, RPT)])


def _edge_pass(feat, el0, el1, er0, er1, src, dst):
    fn = pl.kernel(
        _edge_body,
        out_type=[
            jax.ShapeDtypeStruct((2, N, D), jnp.float32),
            jax.ShapeDtypeStruct((4 * N,), jnp.float32),
        ],
        mesh=plsc.VectorSubcoreMesh(core_axis_name="c", subcore_axis_name="s"),
        compiler_params=pltpu.CompilerParams(needs_layout_passes=False),
        scratch_types=[
            pltpu.VMEM_SHARED((NP, D), jnp.float32),
            pltpu.VMEM_SHARED((NP,), jnp.float32),
            pltpu.VMEM_SHARED((NP,), jnp.float32),
            pltpu.VMEM((CH,), jnp.int32),
            pltpu.VMEM((CH,), jnp.int32),
            pltpu.VMEM((CH,), jnp.float32),
            pltpu.VMEM((CH,), jnp.float32),
            pltpu.VMEM((CH,), jnp.float32),
            pltpu.VMEM((CH,), jnp.float32),
            pltpu.VMEM((CH, D), jnp.float32),
            pltpu.VMEM((CH,), jnp.float32),
            pltpu.VMEM((CH,), jnp.float32),
            pltpu.VMEM((RB, D), jnp.float32),
            pltpu.VMEM((ZD,), jnp.float32),
        ],
    )
    return fn(feat, el0, el1, er0, er1, src, dst)


def _dense2_body(acc_ref, den_ref, W2_ref, feat2_ref):
    a = acc_ref[0] + acc_ref[1]
    d0 = den_ref[:, 0:1] + den_ref[:, 2:3]
    d1 = den_ref[:, 1:2] + den_ref[:, 3:4]
    o0 = a[:, :Dh] / (d0 + 1e-9)
    o1 = a[:, Dh:] / (d1 + 1e-9)
    o = jnp.concatenate([o0, o1], axis=1)
    h = jnp.where(o > 0, o, jnp.exp(o) - 1.0)
    feat2_ref[...] = jnp.dot(h, W2_ref[...], preferred_element_type=jnp.float32)


def _dense3_body(acc_ref, den_ref, chan_ref,
                 Wc13_ref, b13_ref, Wc31_ref, b31_ref,
                 Wm1_ref, bm1_ref, Wm2_ref, bm2_ref,
                 Wf1a_ref, Wf1b_ref, bf1_ref, Wf2_ref, bf2_ref,
                 Wa1a_ref, Wa1b_ref, ba1_ref, Wa2_ref, ba2_ref,
                 fused_ref, att_ref):
    a = acc_ref[0] + acc_ref[1]
    d0 = den_ref[:, 0:1] + den_ref[:, 2:3]
    d1 = den_ref[:, 1:2] + den_ref[:, 3:4]
    o0 = a[:, :Dh] / (d0 + 1e-9)
    o1 = a[:, Dh:] / (d1 + 1e-9)
    o = jnp.concatenate([o0, o1], axis=1)
    sp = jnp.where(o > 0, o, jnp.exp(o) - 1.0)
    chan = chan_ref[...]
    f13 = jnp.dot(sp, Wc13_ref[...], preferred_element_type=jnp.float32) + b13_ref[...]
    f31 = jnp.dot(sp, Wc31_ref[...], preferred_element_type=jnp.float32) + b31_ref[...]
    fc = jnp.dot(jax.nn.relu(jnp.dot(chan, Wm1_ref[...], preferred_element_type=jnp.float32) + bm1_ref[...]),
                 Wm2_ref[...], preferred_element_type=jnp.float32) + bm2_ref[...]
    p = jax.nn.relu(jnp.dot(f13 * f31, Wf1a_ref[...], preferred_element_type=jnp.float32)
                    + jnp.dot(fc * fc, Wf1b_ref[...], preferred_element_type=jnp.float32)
                    + bf1_ref[...])
    fused_ref[...] = jnp.dot(p, Wf2_ref[...], preferred_element_type=jnp.float32) + bf2_ref[...]
    am = jax.nn.relu(jnp.dot(sp, Wa1a_ref[...], preferred_element_type=jnp.float32)
                     + jnp.dot(chan, Wa1b_ref[...], preferred_element_type=jnp.float32)
                     + ba1_ref[...])
    af = jnp.dot(am, Wa2_ref[...], preferred_element_type=jnp.float32) + ba2_ref[...]
    att_ref[...] = jnp.concatenate([sp * af[:, :D], chan * af[:, D:]], axis=1)


def _scores_body(att_ref, ws_ref, bs_ref, scores_ref):
    sc = jnp.sum(att_ref[...] * ws_ref[...], axis=1) + bs_ref[0, 0]
    scores_ref[...] = sc[None, :]


def _pooled_body(att_hbm, idx_hbm, out_hbm, idxb, prow):
    c = lax.axis_index("c")
    s = lax.axis_index("s")
    wid = s * 2 + c
    for t in range(PPW // PC):
        off = wid * PPW + t * PC
        pltpu.sync_copy(idx_hbm.at[pl.ds(off, PC)], idxb)
        pltpu.sync_copy(att_hbm.at[idxb], prow)
        pltpu.sync_copy(prow, out_hbm.at[pl.ds(off, PC)])


def _pooled_gather(att, idxp):
    fn = pl.kernel(
        _pooled_body,
        out_type=jax.ShapeDtypeStruct((KP, 2 * D), jnp.float32),
        mesh=plsc.VectorSubcoreMesh(core_axis_name="c", subcore_axis_name="s"),
        compiler_params=pltpu.CompilerParams(needs_layout_passes=False),
        scratch_types=[
            pltpu.VMEM((PC,), jnp.int32),
            pltpu.VMEM((PC, 2 * D), jnp.float32),
        ],
    )
    return fn(att, idxp)


BN = 1000
GRID = N // BN


def _rep(shape):
    return pl.BlockSpec(shape, lambda i: (0,) * len(shape))


def kernel(x, edge_index, W1, al1, ar1, W2, al2, ar2, Wg1, bg1, Wg2, bg2, Wg3, bg3, Wg4, bg4, Wc13, b13, Wc31, b31, Wm1, bm1, Wm2, bm2, Wf1, bf1, Wf2, bf2, Wa1, ba1, Wa2, ba2, ws, bs):
    src = edge_index[0]
    dst = edge_index[1]
    pad = EPAD - E
    srcp = jnp.concatenate([src, jnp.zeros((pad,), jnp.int32)])
    dstp = jnp.concatenate([dst, jnp.full((pad,), N, jnp.int32)])

    feat1, chan = pl.pallas_call(
        _dense1_body,
        grid=(GRID,),
        in_specs=[
            pl.BlockSpec((BN, D), lambda i: (i, 0)),
            _rep((D, D)),
            _rep((D, D)), _rep((1, D)), _rep((D, D)), _rep((1, D)),
            _rep((D, D)), _rep((1, D)), _rep((D, D)), _rep((1, D)),
        ],
        out_specs=[
            pl.BlockSpec((BN, D), lambda i: (i, 0)),
            pl.BlockSpec((BN, D), lambda i: (i, 0)),
        ],
        out_shape=[
            jax.ShapeDtypeStruct((N, D), jnp.float32),
            jax.ShapeDtypeStruct((N, D), jnp.float32),
        ],
    )(x, W1,
      Wg1, bg1[None, :], Wg2, bg2[None, :], Wg3, bg3[None, :], Wg4, bg4[None, :])
    elr1 = _elr(feat1, al1, ar1)

    acc1, den1f = _edge_pass(feat1, elr1[0], elr1[1], elr1[2], elr1[3], srcp, dstp)
    den1 = den1f.reshape(4, N).T

    feat2 = pl.pallas_call(
        _dense2_body,
        grid=(GRID,),
        in_specs=[
            pl.BlockSpec((2, BN, D), lambda i: (0, i, 0)),
            pl.BlockSpec((BN, 4), lambda i: (i, 0)),
            _rep((D, D)),
        ],
        out_specs=pl.BlockSpec((BN, D), lambda i: (i, 0)),
        out_shape=jax.ShapeDtypeStruct((N, D), jnp.float32),
    )(acc1, den1, W2)
    elr2 = _elr(feat2, al2, ar2)

    acc2, den2f = _edge_pass(feat2, elr2[0], elr2[1], elr2[2], elr2[3], srcp, dstp)
    den2 = den2f.reshape(4, N).T

    fused, att = pl.pallas_call(
        _dense3_body,
        grid=(GRID,),
        in_specs=[
            pl.BlockSpec((2, BN, D), lambda i: (0, i, 0)),
            pl.BlockSpec((BN, 4), lambda i: (i, 0)),
            pl.BlockSpec((BN, D), lambda i: (i, 0)),
            _rep((D, D)), _rep((1, D)), _rep((D, D)), _rep((1, D)),
            _rep((D, D)), _rep((1, D)), _rep((D, D)), _rep((1, D)),
            _rep((D, D)), _rep((D, D)), _rep((1, D)), _rep((D, D)), _rep((1, D)),
            _rep((D, D)), _rep((D, D)), _rep((1, D)), _rep((D, 2 * D)), _rep((1, 2 * D)),
        ],
        out_specs=[
            pl.BlockSpec((BN, D), lambda i: (i, 0)),
            pl.BlockSpec((BN, 2 * D), lambda i: (i, 0)),
        ],
        out_shape=[
            jax.ShapeDtypeStruct((N, D), jnp.float32),
            jax.ShapeDtypeStruct((N, 2 * D), jnp.float32),
        ],
    )(acc2, den2, chan,
      Wc13[:, :, 1].T, b13[None, :], Wc31[:, :, 1].T, b31[None, :],
      Wm1, bm1[None, :], Wm2, bm2[None, :],
      Wf1[:D], Wf1[D:], bf1[None, :], Wf2, bf2[None, :],
      Wa1[:D], Wa1[D:], ba1[None, :], Wa2, ba2[None, :])

    scores = pl.pallas_call(
        _scores_body,
        out_shape=jax.ShapeDtypeStruct((1, N), jnp.float32),
    )(att, ws[:, 0][None, :], bs[None, :])

    _, idx = lax.top_k(scores[0], K)
    idxp = jnp.concatenate([idx, jnp.zeros((KP - K,), jnp.int32)])
    pooled = _pooled_gather(att, idxp)[:K]
    return pooled, fused, idx


# speedup vs baseline: 32.1173x; 1.1890x over previous
import functools

import jax
import jax.numpy as jnp
from jax import lax
from jax.experimental import pallas as pl
from jax.experimental.pallas import tpu as pltpu
from jax.experimental.pallas import tpu_sc as plsc

N, E, D = 10000, 320000, 128
H, Dh = 2, 64
NEG = 0.2
K = int(0.5 * 0.9 * N)

NW = 32
CH = 128
CHUNKS = 79
EPW = CH * CHUNKS
EPAD = NW * EPW
NP = N + 8
WT = 10
RPT = N // WT
RB = 40
ZD = 2000

KP = 4608
PPW = KP // NW
PC = 72


def _dense1_body(x_ref, W1_ref,
                 Wg1_ref, bg1_ref, Wg2_ref, bg2_ref, Wg3_ref, bg3_ref,
                 Wg4_ref, bg4_ref, feat_ref, chan_ref):
    x = x_ref[...]
    feat = jnp.dot(x, W1_ref[...], preferred_element_type=jnp.float32)
    feat_ref[...] = feat
    c = jnp.dot(jax.nn.relu(jnp.dot(x, Wg1_ref[...], preferred_element_type=jnp.float32) + bg1_ref[...]),
                Wg2_ref[...], preferred_element_type=jnp.float32) + bg2_ref[...]
    mx = jnp.max(c, axis=1, keepdims=True)
    av = jnp.mean(c, axis=1, keepdims=True)
    comb = jax.nn.sigmoid(mx) * c + jax.nn.sigmoid(av) * c
    chan = jnp.dot(jax.nn.relu(jnp.dot(comb, Wg3_ref[...], preferred_element_type=jnp.float32) + bg3_ref[...]),
                   Wg4_ref[...], preferred_element_type=jnp.float32) + bg4_ref[...]
    chan_ref[...] = chan


def _elr_body(feat_ref, al0_ref, al1_ref, ar0_ref, ar1_ref, elr_ref):
    feat = feat_ref[...]
    f0 = feat[:, :Dh]
    f1 = feat[:, Dh:]
    el0 = jnp.sum(f0 * al0_ref[...], axis=1)
    el1 = jnp.sum(f1 * al1_ref[...], axis=1)
    er0 = jnp.sum(f0 * ar0_ref[...], axis=1)
    er1 = jnp.sum(f1 * ar1_ref[...], axis=1)
    elr_ref[...] = jnp.stack([el0, el1, er0, er1], axis=0)


def _elr(feat, al, ar):
    return pl.pallas_call(
        _elr_body,
        out_shape=jax.ShapeDtypeStruct((4, N), jnp.float32),
    )(feat, al[0, 0][None, :], al[0, 1][None, :], ar[0, 0][None, :], ar[0, 1][None, :])


def _edge_body(feat_hbm, el0_hbm, el1_hbm, er0_hbm, er1_hbm, src_hbm, dst_hbm,
               acc_out, den_out,
               acc_sp, den0_sp, den1_sp,
               srcb, dstb, el0c, el1c, er0c, er1c, rows, ee0b, ee1b, zb, zd):
    c = lax.axis_index("c")
    s = lax.axis_index("s")
    wid = s * 2 + c

    def _zrow(i, _):
        for k in range(8):
            zb[i, pl.ds(k * 16, 16)] = jnp.zeros((16,), jnp.float32)
        return 0
    lax.fori_loop(0, RB, _zrow, 0)

    def _zd16(i, _):
        zd[pl.ds(i * 16, 16)] = jnp.zeros((16,), jnp.float32)
        return 0
    lax.fori_loop(0, ZD // 16, _zd16, 0)

    @pl.when(s < WT)
    def _zero_acc():
        def _zi(i, _):
            pltpu.sync_copy(zb, acc_sp.at[pl.ds(s * RPT + i * RB, RB)])
            return 0
        lax.fori_loop(0, RPT // RB, _zi, 0)
        pltpu.sync_copy(zd.at[pl.ds(0, RPT)], den0_sp.at[pl.ds(s * RPT, RPT)])
        pltpu.sync_copy(zd.at[pl.ds(0, RPT)], den1_sp.at[pl.ds(s * RPT, RPT)])

    plsc.subcore_barrier()

    def _chunk(ch, _):
        off = wid * EPW + ch * CH
        pltpu.sync_copy(src_hbm.at[pl.ds(off, CH)], srcb)
        pltpu.sync_copy(dst_hbm.at[pl.ds(off, CH)], dstb)
        pltpu.sync_copy(el0_hbm.at[srcb], el0c)
        pltpu.sync_copy(el1_hbm.at[srcb], el1c)
        pltpu.sync_copy(er0_hbm.at[dstb], er0c)
        pltpu.sync_copy(er1_hbm.at[dstb], er1c)
        pltpu.sync_copy(feat_hbm.at[srcb], rows)

        def _grp(g, _):
            sl16 = pl.ds(g * 16, 16)
            e0 = el0c[sl16] + er0c[sl16]
            e0 = jnp.where(e0 >= 0, e0, NEG * e0)
            ee0b[sl16] = jnp.exp(e0)
            e1 = el1c[sl16] + er1c[sl16]
            e1 = jnp.where(e1 >= 0, e1, NEG * e1)
            ee1b[sl16] = jnp.exp(e1)
            return 0
        lax.fori_loop(0, CH // 16, _grp, 0)

        def _scale(g, _):
            v0 = ee0b[pl.ds(g * 16, 16)]
            v1 = ee1b[pl.ds(g * 16, 16)]
            for j in range(16):
                s0 = v0[j]
                s1 = v1[j]
                for k in range(4):
                    sl = pl.ds(k * 16, 16)
                    rows[g * 16 + j, sl] = rows[g * 16 + j, sl] * s0
                for k in range(4, 8):
                    sl = pl.ds(k * 16, 16)
                    rows[g * 16 + j, sl] = rows[g * 16 + j, sl] * s1
            return 0
        lax.fori_loop(0, CH // 16, _scale, 0)

        pltpu.sync_copy(rows, acc_sp.at[dstb], add=True)
        pltpu.sync_copy(ee0b, den0_sp.at[dstb], add=True)
        pltpu.sync_copy(ee1b, den1_sp.at[dstb], add=True)
        return 0
    lax.fori_loop(0, CHUNKS, _chunk, 0)

    plsc.subcore_barrier()

    @pl.when(s < WT)
    def _write_acc():
        def _wi(i, _):
            r0 = s * RPT + i * RB
            pltpu.sync_copy(acc_sp.at[pl.ds(r0, RB)], zb)
            pltpu.sync_copy(zb, acc_out.at[c, pl.ds(r0, RB)])
            return 0
        lax.fori_loop(0, RPT // RB, _wi, 0)
        pltpu.sync_copy(den0_sp.at[pl.ds(s * RPT, RPT)], zd.at[pl.ds(0, RPT)])
        pltpu.sync_copy(zd.at[pl.ds(0, RPT)],
                        den_out.at[pl.ds(c * 2 * N + s * RPT, RPT)])
        pltpu.sync_copy(den1_sp.at[pl.ds(s * RPT, RPT)], zd.at[pl.ds(0, RPT)])
        pltpu.sync_copy(zd.at[pl.ds(0, RPT)],
                        den_out.at[pl.ds(c * 2 * N + N + s * RPT, RPT)])


def _edge_pass(feat, el0, el1, er0, er1, src, dst):
    fn = pl.kernel(
        _edge_body,
        out_type=[
            jax.ShapeDtypeStruct((2, N, D), jnp.float32),
            jax.ShapeDtypeStruct((4 * N,), jnp.float32),
        ],
        mesh=plsc.VectorSubcoreMesh(core_axis_name="c", subcore_axis_name="s"),
        compiler_params=pltpu.CompilerParams(needs_layout_passes=False),
        scratch_types=[
            pltpu.VMEM_SHARED((NP, D), jnp.float32),
            pltpu.VMEM_SHARED((NP,), jnp.float32),
            pltpu.VMEM_SHARED((NP,), jnp.float32),
            pltpu.VMEM((CH,), jnp.int32),
            pltpu.VMEM((CH,), jnp.int32),
            pltpu.VMEM((CH,), jnp.float32),
            pltpu.VMEM((CH,), jnp.float32),
            pltpu.VMEM((CH,), jnp.float32),
            pltpu.VMEM((CH,), jnp.float32),
            pltpu.VMEM((CH, D), jnp.float32),
            pltpu.VMEM((CH,), jnp.float32),
            pltpu.VMEM((CH,), jnp.float32),
            pltpu.VMEM((RB, D), jnp.float32),
            pltpu.VMEM((ZD,), jnp.float32),
        ],
    )
    return fn(feat, el0, el1, er0, er1, src, dst)


def _dense2_body(acc_ref, den_ref, W2_ref, feat2_ref):
    a = acc_ref[0] + acc_ref[1]
    d0 = den_ref[:, 0:1] + den_ref[:, 2:3]
    d1 = den_ref[:, 1:2] + den_ref[:, 3:4]
    o0 = a[:, :Dh] / (d0 + 1e-9)
    o1 = a[:, Dh:] / (d1 + 1e-9)
    o = jnp.concatenate([o0, o1], axis=1)
    h = jnp.where(o > 0, o, jnp.exp(o) - 1.0)
    feat2_ref[...] = jnp.dot(h, W2_ref[...], preferred_element_type=jnp.float32)


def _dense3_body(acc_ref, den_ref, chan_ref,
                 Wc13_ref, b13_ref, Wc31_ref, b31_ref,
                 Wm1_ref, bm1_ref, Wm2_ref, bm2_ref,
                 Wf1a_ref, Wf1b_ref, bf1_ref, Wf2_ref, bf2_ref,
                 Wa1a_ref, Wa1b_ref, ba1_ref, Wa2_ref, ba2_ref,
                 fused_ref, att_ref):
    a = acc_ref[0] + acc_ref[1]
    d0 = den_ref[:, 0:1] + den_ref[:, 2:3]
    d1 = den_ref[:, 1:2] + den_ref[:, 3:4]
    o0 = a[:, :Dh] / (d0 + 1e-9)
    o1 = a[:, Dh:] / (d1 + 1e-9)
    o = jnp.concatenate([o0, o1], axis=1)
    sp = jnp.where(o > 0, o, jnp.exp(o) - 1.0)
    chan = chan_ref[...]
    f13 = jnp.dot(sp, Wc13_ref[...], preferred_element_type=jnp.float32) + b13_ref[...]
    f31 = jnp.dot(sp, Wc31_ref[...], preferred_element_type=jnp.float32) + b31_ref[...]
    fc = jnp.dot(jax.nn.relu(jnp.dot(chan, Wm1_ref[...], preferred_element_type=jnp.float32) + bm1_ref[...]),
                 Wm2_ref[...], preferred_element_type=jnp.float32) + bm2_ref[...]
    p = jax.nn.relu(jnp.dot(f13 * f31, Wf1a_ref[...], preferred_element_type=jnp.float32)
                    + jnp.dot(fc * fc, Wf1b_ref[...], preferred_element_type=jnp.float32)
                    + bf1_ref[...])
    fused_ref[...] = jnp.dot(p, Wf2_ref[...], preferred_element_type=jnp.float32) + bf2_ref[...]
    am = jax.nn.relu(jnp.dot(sp, Wa1a_ref[...], preferred_element_type=jnp.float32)
                     + jnp.dot(chan, Wa1b_ref[...], preferred_element_type=jnp.float32)
                     + ba1_ref[...])
    af = jnp.dot(am, Wa2_ref[...], preferred_element_type=jnp.float32) + ba2_ref[...]
    att_ref[...] = jnp.concatenate([sp * af[:, :D], chan * af[:, D:]], axis=1)


def _scores_body(att_ref, ws_ref, bs_ref, scores_ref):
    sc = jnp.sum(att_ref[...] * ws_ref[...], axis=1) + bs_ref[0, 0]
    scores_ref[...] = sc[None, :]


def _pooled_body(att_hbm, idx_hbm, out_hbm, idxb, prow):
    c = lax.axis_index("c")
    s = lax.axis_index("s")
    wid = s * 2 + c
    for t in range(PPW // PC):
        off = wid * PPW + t * PC
        pltpu.sync_copy(idx_hbm.at[pl.ds(off, PC)], idxb)
        pltpu.sync_copy(att_hbm.at[idxb], prow)
        pltpu.sync_copy(prow, out_hbm.at[pl.ds(off, PC)])


def _pooled_gather(att, idxp):
    fn = pl.kernel(
        _pooled_body,
        out_type=jax.ShapeDtypeStruct((KP, 2 * D), jnp.float32),
        mesh=plsc.VectorSubcoreMesh(core_axis_name="c", subcore_axis_name="s"),
        compiler_params=pltpu.CompilerParams(needs_layout_passes=False),
        scratch_types=[
            pltpu.VMEM((PC,), jnp.int32),
            pltpu.VMEM((PC, 2 * D), jnp.float32),
        ],
    )
    return fn(att, idxp)


BN = 1000
GRID = N // BN


def _rep(shape):
    return pl.BlockSpec(shape, lambda i: (0,) * len(shape))


def kernel(x, edge_index, W1, al1, ar1, W2, al2, ar2, Wg1, bg1, Wg2, bg2, Wg3, bg3, Wg4, bg4, Wc13, b13, Wc31, b31, Wm1, bm1, Wm2, bm2, Wf1, bf1, Wf2, bf2, Wa1, ba1, Wa2, ba2, ws, bs):
    src = edge_index[0]
    dst = edge_index[1]
    pad = EPAD - E
    srcp = jnp.concatenate([src, jnp.zeros((pad,), jnp.int32)])
    dstp = jnp.concatenate([dst, jnp.full((pad,), N, jnp.int32)])

    feat1, chan = pl.pallas_call(
        _dense1_body,
        grid=(GRID,),
        in_specs=[
            pl.BlockSpec((BN, D), lambda i: (i, 0)),
            _rep((D, D)),
            _rep((D, D)), _rep((1, D)), _rep((D, D)), _rep((1, D)),
            _rep((D, D)), _rep((1, D)), _rep((D, D)), _rep((1, D)),
        ],
        out_specs=[
            pl.BlockSpec((BN, D), lambda i: (i, 0)),
            pl.BlockSpec((BN, D), lambda i: (i, 0)),
        ],
        out_shape=[
            jax.ShapeDtypeStruct((N, D), jnp.float32),
            jax.ShapeDtypeStruct((N, D), jnp.float32),
        ],
    )(x, W1,
      Wg1, bg1[None, :], Wg2, bg2[None, :], Wg3, bg3[None, :], Wg4, bg4[None, :])
    elr1 = _elr(feat1, al1, ar1)

    acc1, den1f = _edge_pass(feat1, elr1[0], elr1[1], elr1[2], elr1[3], srcp, dstp)
    den1 = den1f.reshape(4, N).T

    feat2 = pl.pallas_call(
        _dense2_body,
        grid=(GRID,),
        in_specs=[
            pl.BlockSpec((2, BN, D), lambda i: (0, i, 0)),
            pl.BlockSpec((BN, 4), lambda i: (i, 0)),
            _rep((D, D)),
        ],
        out_specs=pl.BlockSpec((BN, D), lambda i: (i, 0)),
        out_shape=jax.ShapeDtypeStruct((N, D), jnp.float32),
    )(acc1, den1, W2)
    elr2 = _elr(feat2, al2, ar2)

    acc2, den2f = _edge_pass(feat2, elr2[0], elr2[1], elr2[2], elr2[3], srcp, dstp)
    den2 = den2f.reshape(4, N).T

    fused, att = pl.pallas_call(
        _dense3_body,
        grid=(GRID,),
        in_specs=[
            pl.BlockSpec((2, BN, D), lambda i: (0, i, 0)),
            pl.BlockSpec((BN, 4), lambda i: (i, 0)),
            pl.BlockSpec((BN, D), lambda i: (i, 0)),
            _rep((D, D)), _rep((1, D)), _rep((D, D)), _rep((1, D)),
            _rep((D, D)), _rep((1, D)), _rep((D, D)), _rep((1, D)),
            _rep((D, D)), _rep((D, D)), _rep((1, D)), _rep((D, D)), _rep((1, D)),
            _rep((D, D)), _rep((D, D)), _rep((1, D)), _rep((D, 2 * D)), _rep((1, 2 * D)),
        ],
        out_specs=[
            pl.BlockSpec((BN, D), lambda i: (i, 0)),
            pl.BlockSpec((BN, 2 * D), lambda i: (i, 0)),
        ],
        out_shape=[
            jax.ShapeDtypeStruct((N, D), jnp.float32),
            jax.ShapeDtypeStruct((N, 2 * D), jnp.float32),
        ],
    )(acc2, den2, chan,
      Wc13[:, :, 1].T, b13[None, :], Wc31[:, :, 1].T, b31[None, :],
      Wm1, bm1[None, :], Wm2, bm2[None, :],
      Wf1[:D], Wf1[D:], bf1[None, :], Wf2, bf2[None, :],
      Wa1[:D], Wa1[D:], ba1[None, :], Wa2, ba2[None, :])

    scores = pl.pallas_call(
        _scores_body,
        out_shape=jax.ShapeDtypeStruct((1, N), jnp.float32),
    )(att, ws[:, 0][None, :], bs[None, :])

    _, idx = lax.top_k(scores[0], K)
    idxp = jnp.concatenate([idx, jnp.zeros((KP - K,), jnp.int32)])
    pooled = _pooled_gather(att, idxp)[:K]
    return pooled, fused, idx
